# baseline (speedup 1.0000x reference)
import functools

import jax
import jax.numpy as jnp
from jax import lax
from jax.experimental import pallas as pl
from jax.experimental.pallas import tpu as pltpu
from jax.experimental.pallas import tpu_sc as plsc

_F = 100
_D = 64
_LANES = 16
_NWORKERS = 32
_CHUNK = 320
_NPAIR_PAD = _NWORKERS * _CHUNK


def _sc_pair_dots(vs2, fields_pad):
    groups = _CHUNK // _LANES
    mesh = plsc.VectorSubcoreMesh(core_axis_name="c", subcore_axis_name="s")

    @functools.partial(
        pl.kernel,
        mesh=mesh,
        compiler_params=pltpu.CompilerParams(
            needs_layout_passes=False, use_tc_tiling_on_sc=False),
        out_type=jax.ShapeDtypeStruct((_NPAIR_PAD,), jnp.float32),
        scratch_types=[
            pltpu.VMEM((128,), jnp.int32),
            pltpu.VMEM((3, 128), jnp.int32),
            pltpu.VMEM((3, 128), jnp.int32),
            pltpu.VMEM((3 * 128, _D), jnp.float32),
            pltpu.VMEM((3 * 128, _D), jnp.float32),
            pltpu.VMEM((_CHUNK,), jnp.float32),
            pltpu.VMEM((_LANES, 17), jnp.float32),
            pltpu.VMEM_SHARED((2600, _D), jnp.float32),
            pltpu.SemaphoreType.DMA,
            pltpu.SemaphoreType.DMA,
        ],
    )
    def k(vs2_hbm, fields_hbm, m_hbm, fields_v, ia_v, ib_v, ra_v, rb_v, m_v,
          t_v, vs2_sh, sem_a, sem_b):
        wid = lax.axis_index("s") * 2 + lax.axis_index("c")
        base = wid * _CHUNK
        @pl.when(lax.axis_index("s") == 0)
        def _():
            pltpu.sync_copy(vs2_hbm, vs2_sh)
        pltpu.sync_copy(fields_hbm, fields_v)
        lane = lax.iota(jnp.int32, _LANES)

        def fill(g, carry):
            p = jnp.minimum(base + g * _LANES + lane, _F * _F - 1)
            i = lax.div(p, jnp.int32(_F))
            j = p - i * _F
            fi = plsc.load_gather(fields_v, [i])
            fj = plsc.load_gather(fields_v, [j])
            row = lax.div(g, jnp.int32(8))
            col = lax.rem(g, jnp.int32(8)) * _LANES
            ia_v[row, pl.ds(col, _LANES)] = fi * _F + j
            ib_v[row, pl.ds(col, _LANES)] = fj * _F + i
            return carry

        with jax.named_scope("sc_fill"):
            lax.fori_loop(0, 24, fill, 0)

        plsc.subcore_barrier()
        with jax.named_scope("sc_gather"):
         for r in range(3):
            pltpu.async_copy(vs2_sh.at[ia_v.at[r]],
                             ra_v.at[pl.ds(r * 128, 128), :], sem_a)
            pltpu.async_copy(vs2_sh.at[ib_v.at[r]],
                             rb_v.at[pl.ds(r * 128, 128), :], sem_b)
         for r in range(3):
            pltpu.make_async_copy(vs2_hbm.at[pl.ds(0, 128)],
                                  ra_v.at[pl.ds(r * 128, 128), :],
                                  sem_a).wait()
            pltpu.make_async_copy(vs2_hbm.at[pl.ds(0, 128)],
                                  rb_v.at[pl.ds(r * 128, 128), :],
                                  sem_b).wait()

        def dots(g, carry):
            p0 = g * _LANES
            for r in range(_LANES):
                acc = jnp.zeros((_LANES,), jnp.float32)
                for c in range(_D // _LANES):
                    a = ra_v[p0 + r, pl.ds(c * _LANES, _LANES)]
                    bb = rb_v[p0 + r, pl.ds(c * _LANES, _LANES)]
                    acc = acc + a * bb
                t_v[r, pl.ds(0, _LANES)] = acc
            m = jnp.zeros((_LANES,), jnp.float32)
            for l in range(_LANES):
                m = m + plsc.load_gather(
                    t_v, [lane, jnp.full((_LANES,), l, jnp.int32)])
            m_v[pl.ds(p0, _LANES)] = m
            return carry

        with jax.named_scope("sc_dots"):
            lax.fori_loop(0, groups, dots, 0)
        pltpu.sync_copy(m_v, m_hbm.at[pl.ds(base, _CHUNK)])

    return k(vs2, fields_pad)


def _tc_stream(x, wcol, eye):
    bsz, f = x.shape
    blk = 2048
    nblk = bsz // blk
    rows = blk // 128

    def body(x_hbm, wcol_ref, eye_ref, lin_ref, s_ref, buf, sems):
        step = pl.program_id(0)
        slot = lax.rem(step, 2)
        nslot = lax.rem(step + 1, 2)

        @pl.when(step == 0)
        def _():
            s_ref[...] = jnp.zeros_like(s_ref)
            pltpu.make_async_copy(
                x_hbm.at[pl.ds(0, blk), :], buf.at[0], sems.at[0]).start()

        @pl.when(step + 1 < nblk)
        def _():
            pltpu.make_async_copy(
                x_hbm.at[pl.ds((step + 1) * blk, blk), :], buf.at[nslot],
                sems.at[nslot]).start()

        pltpu.make_async_copy(
            x_hbm.at[pl.ds(0, blk), :], buf.at[slot], sems.at[slot]).wait()
        xb = buf[slot]
        s_ref[...] += lax.dot_general(
            xb, xb, (((0,), (0,)), ((), ())),
            preferred_element_type=jnp.float32)
        lincol = lax.dot_general(
            xb, wcol_ref[...], (((1,), (0,)), ((), ())),
            preferred_element_type=jnp.float32)
        parts = [lax.slice(lincol, (q * 128, 0), ((q + 1) * 128, 1))
                 for q in range(rows)]
        lin_ref[...] = jnp.concatenate(
            [lax.dot_general(
                jnp.concatenate(parts[g * 8:(g + 1) * 8], axis=1),
                eye_ref[...], (((0,), (0,)), ((), ())),
                preferred_element_type=jnp.float32)
             for g in range(rows // 8)], axis=0)

    return pl.pallas_call(
        body,
        grid=(nblk,),
        in_specs=[
            pl.BlockSpec(memory_space=pltpu.HBM),
            pl.BlockSpec((f, 1), lambda bi: (0, 0)),
            pl.BlockSpec((128, 128), lambda bi: (0, 0)),
        ],
        out_specs=[
            pl.BlockSpec((rows, 128), lambda bi: (bi, 0)),
            pl.BlockSpec((f, f), lambda bi: (0, 0)),
        ],
        out_shape=[
            jax.ShapeDtypeStruct((bsz // 128, 128), jnp.float32),
            jax.ShapeDtypeStruct((f, f), jnp.float32),
        ],
        scratch_shapes=[
            pltpu.VMEM((2, blk, f), jnp.float32),
            pltpu.SemaphoreType.DMA((2,)),
        ],
        compiler_params=pltpu.CompilerParams(
            fuse_transposed_lhs_in_matmul=True),
    )(pltpu.with_memory_space_constraint(x, pltpu.HBM), wcol, eye)


def _tc_finish(lin2d, m, s, bias):
    def body(lin_ref, m_ref, s_ref, b_ref, o_ref):
        mm = m_ref[...]
        ss = s_ref[0:mm.shape[0], 0:mm.shape[1]]
        ii = lax.broadcasted_iota(jnp.int32, mm.shape, 0)
        jj = lax.broadcasted_iota(jnp.int32, mm.shape, 1)
        s2 = jnp.sum(jnp.where(jj > ii, mm * ss, 0.0))
        o_ref[...] = jax.nn.sigmoid(lin_ref[...] + (s2 + b_ref[0, 0]))

    return pl.pallas_call(
        body,
        out_shape=jax.ShapeDtypeStruct(lin2d.shape, jnp.float32),
    )(lin2d, m, s, bias)


def kernel(input, fields_dict, W, b, vs):
    fn, f, d = vs.shape
    bsz = input.shape[0]
    vs2 = vs.reshape(fn * f, d)
    fields_pad = jnp.zeros((128,), jnp.int32).at[:f].set(fields_dict)

    m_flat = _sc_pair_dots(vs2, fields_pad)
    m = m_flat[: f * f].reshape(f, f)

    x128 = lax.dynamic_update_slice(
        jnp.zeros((bsz, 128), jnp.float32), input, (0, 0))
    w128 = jnp.concatenate(
        [W.reshape(f, 1), jnp.zeros((128 - f, 1), jnp.float32)], axis=0)
    lin2d, s = _tc_stream(x128, w128, jnp.eye(128, dtype=jnp.float32))
    out2d = _tc_finish(lin2d, m, s, b.reshape(1, 1))
    return out2d.reshape(bsz, 1)


# speedup vs baseline: 1.0468x; 1.0468x over previous
import functools

import jax
import jax.numpy as jnp
from jax import lax
from jax.experimental import pallas as pl
from jax.experimental.pallas import tpu as pltpu
from jax.experimental.pallas import tpu_sc as plsc

_F = 100
_D = 64
_LANES = 16
_NWORKERS = 32
_CHUNK = 320
_NPAIR_PAD = _NWORKERS * _CHUNK


def _sc_pair_dots(vs2, fields_pad):
    groups = _CHUNK // _LANES
    mesh = plsc.VectorSubcoreMesh(core_axis_name="c", subcore_axis_name="s")

    @functools.partial(
        pl.kernel,
        mesh=mesh,
        compiler_params=pltpu.CompilerParams(
            needs_layout_passes=False, use_tc_tiling_on_sc=False),
        out_type=jax.ShapeDtypeStruct((_NPAIR_PAD,), jnp.float32),
        scratch_types=[
            pltpu.VMEM((128,), jnp.int32),
            pltpu.VMEM((3, 128), jnp.int32),
            pltpu.VMEM((3, 128), jnp.int32),
            pltpu.VMEM((3 * 128, _D), jnp.float32),
            pltpu.VMEM((3 * 128, _D), jnp.float32),
            pltpu.VMEM((_CHUNK,), jnp.float32),
            pltpu.VMEM((_LANES, 17), jnp.float32),
            pltpu.VMEM_SHARED((2600, _D), jnp.float32),
            pltpu.SemaphoreType.DMA,
            pltpu.SemaphoreType.DMA,
        ],
    )
    def k(vs2_hbm, fields_hbm, m_hbm, fields_v, ia_v, ib_v, ra_v, rb_v, m_v,
          t_v, vs2_sh, sem_a, sem_b):
        wid = lax.axis_index("s") * 2 + lax.axis_index("c")
        base = wid * _CHUNK
        @pl.when(lax.axis_index("s") == 0)
        def _():
            pltpu.sync_copy(vs2_hbm, vs2_sh)
        pltpu.sync_copy(fields_hbm, fields_v)
        lane = lax.iota(jnp.int32, _LANES)

        def fill(g, carry):
            p = jnp.minimum(base + g * _LANES + lane, _F * _F - 1)
            i = lax.div(p, jnp.int32(_F))
            j = p - i * _F
            fi = plsc.load_gather(fields_v, [i])
            fj = plsc.load_gather(fields_v, [j])
            row = lax.div(g, jnp.int32(8))
            col = lax.rem(g, jnp.int32(8)) * _LANES
            ia_v[row, pl.ds(col, _LANES)] = fi * _F + j
            ib_v[row, pl.ds(col, _LANES)] = fj * _F + i
            return carry

        with jax.named_scope("sc_fill"):
            lax.fori_loop(0, 24, fill, 0)

        plsc.subcore_barrier()
        with jax.named_scope("sc_gather"):
         for r in range(3):
            pltpu.async_copy(vs2_sh.at[ia_v.at[r]],
                             ra_v.at[pl.ds(r * 128, 128), :], sem_a)
            pltpu.async_copy(vs2_sh.at[ib_v.at[r]],
                             rb_v.at[pl.ds(r * 128, 128), :], sem_b)
         for r in range(3):
            pltpu.make_async_copy(vs2_hbm.at[pl.ds(0, 128)],
                                  ra_v.at[pl.ds(r * 128, 128), :],
                                  sem_a).wait()
            pltpu.make_async_copy(vs2_hbm.at[pl.ds(0, 128)],
                                  rb_v.at[pl.ds(r * 128, 128), :],
                                  sem_b).wait()

        def dots(g, carry):
            p0 = g * _LANES
            for r in range(_LANES):
                acc = jnp.zeros((_LANES,), jnp.float32)
                for c in range(_D // _LANES):
                    a = ra_v[p0 + r, pl.ds(c * _LANES, _LANES)]
                    bb = rb_v[p0 + r, pl.ds(c * _LANES, _LANES)]
                    acc = acc + a * bb
                t_v[r, pl.ds(0, _LANES)] = acc
            m = jnp.zeros((_LANES,), jnp.float32)
            for l in range(_LANES):
                m = m + plsc.load_gather(
                    t_v, [lane, jnp.full((_LANES,), l, jnp.int32)])
            m_v[pl.ds(p0, _LANES)] = m
            return carry

        with jax.named_scope("sc_dots"):
            lax.fori_loop(0, groups, dots, 0)
        pltpu.sync_copy(m_v, m_hbm.at[pl.ds(base, _CHUNK)])

    return k(vs2, fields_pad)


def _tc_stream(x, wcol, eye):
    bsz, f = x.shape
    blk = 2048
    nblk = bsz // blk
    rows = blk // 128

    def body(x_hbm, wcol_ref, eye_ref, lin_ref, s_ref, buf, sems):
        step = pl.program_id(0)
        slot = lax.rem(step, 2)
        nslot = lax.rem(step + 1, 2)

        @pl.when(step == 0)
        def _():
            s_ref[...] = jnp.zeros_like(s_ref)
            pltpu.make_async_copy(
                x_hbm.at[pl.ds(0, blk), :], buf.at[0], sems.at[0]).start()

        @pl.when(step + 1 < nblk)
        def _():
            pltpu.make_async_copy(
                x_hbm.at[pl.ds((step + 1) * blk, blk), :], buf.at[nslot],
                sems.at[nslot]).start()

        pltpu.make_async_copy(
            x_hbm.at[pl.ds(0, blk), :], buf.at[slot], sems.at[slot]).wait()
        xb = buf[slot]
        s_ref[...] += lax.dot_general(
            xb, xb, (((0,), (0,)), ((), ())),
            preferred_element_type=jnp.float32)
        lincol = lax.dot_general(
            xb, wcol_ref[...], (((1,), (0,)), ((), ())),
            preferred_element_type=jnp.float32)
        parts = [lax.slice(lincol, (q * 128, 0), ((q + 1) * 128, 1))
                 for q in range(rows)]
        lin_ref[...] = jnp.concatenate(
            [lax.dot_general(
                jnp.concatenate(parts[g * 8:(g + 1) * 8], axis=1),
                eye_ref[...], (((0,), (0,)), ((), ())),
                preferred_element_type=jnp.float32)
             for g in range(rows // 8)], axis=0)

    return pl.pallas_call(
        body,
        grid=(nblk,),
        in_specs=[
            pl.BlockSpec(memory_space=pltpu.HBM),
            pl.BlockSpec((f, 1), lambda bi: (0, 0)),
            pl.BlockSpec((128, 128), lambda bi: (0, 0)),
        ],
        out_specs=[
            pl.BlockSpec((rows, 128), lambda bi: (bi, 0)),
            pl.BlockSpec((f, f), lambda bi: (0, 0)),
        ],
        out_shape=[
            jax.ShapeDtypeStruct((bsz // 128, 128), jnp.float32),
            jax.ShapeDtypeStruct((f, f), jnp.float32),
        ],
        scratch_shapes=[
            pltpu.VMEM((2, blk, f), jnp.float32),
            pltpu.SemaphoreType.DMA((2,)),
        ],
        compiler_params=pltpu.CompilerParams(
            fuse_transposed_lhs_in_matmul=True),
    )(pltpu.with_memory_space_constraint(x, pltpu.HBM), wcol, eye)


def _tc_finish(lin2d, m, s, bias):
    def body(lin_ref, m_ref, s_ref, b_ref, o_ref):
        mm = m_ref[...]
        ss = s_ref[0:mm.shape[0], 0:mm.shape[1]]
        ii = lax.broadcasted_iota(jnp.int32, mm.shape, 0)
        jj = lax.broadcasted_iota(jnp.int32, mm.shape, 1)
        s2 = jnp.sum(jnp.where(jj > ii, mm * ss, 0.0))
        o_ref[...] = jax.nn.sigmoid(lin_ref[...] + (s2 + b_ref[0, 0]))

    return pl.pallas_call(
        body,
        out_shape=jax.ShapeDtypeStruct(lin2d.shape, jnp.float32),
    )(lin2d, m, s, bias)


def kernel(input, fields_dict, W, b, vs):
    fn, f, d = vs.shape
    bsz = input.shape[0]
    vs2 = vs.reshape(fn * f, d)
    fields_pad = jnp.zeros((128,), jnp.int32).at[:f].set(fields_dict)

    m_flat = _sc_pair_dots(vs2, fields_pad)
    m = m_flat[: f * f].reshape(f, f)

    x128 = input @ jnp.eye(f, 128, dtype=jnp.float32)
    w128 = jnp.concatenate(
        [W.reshape(f, 1), jnp.zeros((128 - f, 1), jnp.float32)], axis=0)
    lin2d, s = _tc_stream(x128, w128, jnp.eye(128, dtype=jnp.float32))
    out2d = _tc_finish(lin2d, m, s, b.reshape(1, 1))
    return out2d.reshape(bsz, 1)


# speedup vs baseline: 1.1043x; 1.0550x over previous
import functools

import jax
import jax.numpy as jnp
from jax import lax
from jax.experimental import pallas as pl
from jax.experimental.pallas import tpu as pltpu
from jax.experimental.pallas import tpu_sc as plsc

_F = 100
_D = 64
_LANES = 16
_NWORKERS = 32
_CHUNK = 320
_NPAIR_PAD = _NWORKERS * _CHUNK


def _sc_pair_dots(vs2, fields_pad):
    groups = _CHUNK // _LANES
    mesh = plsc.VectorSubcoreMesh(core_axis_name="c", subcore_axis_name="s")

    @functools.partial(
        pl.kernel,
        mesh=mesh,
        compiler_params=pltpu.CompilerParams(
            needs_layout_passes=False, use_tc_tiling_on_sc=False),
        out_type=jax.ShapeDtypeStruct((_NPAIR_PAD,), jnp.float32),
        scratch_types=[
            pltpu.VMEM((128,), jnp.int32),
            pltpu.VMEM((3, 128), jnp.int32),
            pltpu.VMEM((3, 128), jnp.int32),
            pltpu.VMEM((3 * 128, _D), jnp.float32),
            pltpu.VMEM((3 * 128, _D), jnp.float32),
            pltpu.VMEM((_CHUNK,), jnp.float32),
            pltpu.VMEM((_LANES, 17), jnp.float32),
            pltpu.VMEM_SHARED((2600, _D), jnp.float32),
            pltpu.SemaphoreType.DMA,
            pltpu.SemaphoreType.DMA,
        ],
    )
    def k(vs2_hbm, fields_hbm, m_hbm, fields_v, ia_v, ib_v, ra_v, rb_v, m_v,
          t_v, vs2_sh, sem_a, sem_b):
        wid = lax.axis_index("s") * 2 + lax.axis_index("c")
        base = wid * _CHUNK
        @pl.when(lax.axis_index("s") == 0)
        def _():
            pltpu.sync_copy(vs2_hbm, vs2_sh)
        pltpu.sync_copy(fields_hbm, fields_v)
        lane = lax.iota(jnp.int32, _LANES)

        def fill(g, carry):
            p = jnp.minimum(base + g * _LANES + lane, _F * _F - 1)
            i = lax.div(p, jnp.int32(_F))
            j = p - i * _F
            fi = plsc.load_gather(fields_v, [i])
            fj = plsc.load_gather(fields_v, [j])
            row = lax.div(g, jnp.int32(8))
            col = lax.rem(g, jnp.int32(8)) * _LANES
            ia_v[row, pl.ds(col, _LANES)] = fi * _F + j
            ib_v[row, pl.ds(col, _LANES)] = fj * _F + i
            return carry

        with jax.named_scope("sc_fill"):
            lax.fori_loop(0, 24, fill, 0)

        plsc.subcore_barrier()
        with jax.named_scope("sc_gather"):
         for r in range(3):
            pltpu.async_copy(vs2_sh.at[ia_v.at[r]],
                             ra_v.at[pl.ds(r * 128, 128), :], sem_a)
            pltpu.async_copy(vs2_sh.at[ib_v.at[r]],
                             rb_v.at[pl.ds(r * 128, 128), :], sem_b)
         for r in range(3):
            pltpu.make_async_copy(vs2_hbm.at[pl.ds(0, 128)],
                                  ra_v.at[pl.ds(r * 128, 128), :],
                                  sem_a).wait()
            pltpu.make_async_copy(vs2_hbm.at[pl.ds(0, 128)],
                                  rb_v.at[pl.ds(r * 128, 128), :],
                                  sem_b).wait()

        def dots(g, carry):
            p0 = g * _LANES
            for r in range(_LANES):
                acc = jnp.zeros((_LANES,), jnp.float32)
                for c in range(_D // _LANES):
                    a = ra_v[p0 + r, pl.ds(c * _LANES, _LANES)]
                    bb = rb_v[p0 + r, pl.ds(c * _LANES, _LANES)]
                    acc = acc + a * bb
                t_v[r, pl.ds(0, _LANES)] = acc
            m = jnp.zeros((_LANES,), jnp.float32)
            for l in range(_LANES):
                m = m + plsc.load_gather(
                    t_v, [lane, jnp.full((_LANES,), l, jnp.int32)])
            m_v[pl.ds(p0, _LANES)] = m
            return carry

        with jax.named_scope("sc_dots"):
            lax.fori_loop(0, groups, dots, 0)
        pltpu.sync_copy(m_v, m_hbm.at[pl.ds(base, _CHUNK)])

    return k(vs2, fields_pad)


def _tc_stream(x, wcol):
    bsz, f = x.shape
    blk = 4096
    nblk = bsz // blk
    rows = blk // 128

    def body(x_hbm, wcol_ref, lin_ref, s_ref, buf, sems):
        step = pl.program_id(0)
        ii = lax.broadcasted_iota(jnp.int32, (128, 128), 0)
        jj = lax.broadcasted_iota(jnp.int32, (128, 128), 1)
        eye = jnp.where(ii == jj, 1.0, 0.0).astype(jnp.float32)
        slot = lax.rem(step, 2)
        nslot = lax.rem(step + 1, 2)

        @pl.when(step == 0)
        def _():
            s_ref[...] = jnp.zeros_like(s_ref)
            pltpu.make_async_copy(
                x_hbm.at[pl.ds(0, blk), :], buf.at[0], sems.at[0]).start()

        @pl.when(step + 1 < nblk)
        def _():
            pltpu.make_async_copy(
                x_hbm.at[pl.ds((step + 1) * blk, blk), :], buf.at[nslot],
                sems.at[nslot]).start()

        pltpu.make_async_copy(
            x_hbm.at[pl.ds(0, blk), :], buf.at[slot], sems.at[slot]).wait()
        xb = buf[slot]
        s_ref[...] += lax.dot_general(
            xb, xb, (((0,), (0,)), ((), ())),
            preferred_element_type=jnp.float32)
        lincol = lax.dot_general(
            xb, wcol_ref[...], (((1,), (0,)), ((), ())),
            preferred_element_type=jnp.float32)
        parts = [lax.slice(lincol, (q * 128, 0), ((q + 1) * 128, 1))
                 for q in range(rows)]
        lin_ref[...] = jnp.concatenate(
            [lax.dot_general(
                jnp.concatenate(parts[g * 8:(g + 1) * 8], axis=1),
                eye, (((0,), (0,)), ((), ())),
                preferred_element_type=jnp.float32)
             for g in range(rows // 8)], axis=0)

    return pl.pallas_call(
        body,
        grid=(nblk,),
        in_specs=[
            pl.BlockSpec(memory_space=pltpu.HBM),
            pl.BlockSpec((f, 1), lambda bi: (0, 0)),
        ],
        out_specs=[
            pl.BlockSpec((rows, 128), lambda bi: (bi, 0)),
            pl.BlockSpec((f, f), lambda bi: (0, 0)),
        ],
        out_shape=[
            jax.ShapeDtypeStruct((bsz // 128, 128), jnp.float32),
            jax.ShapeDtypeStruct((f, f), jnp.float32),
        ],
        scratch_shapes=[
            pltpu.VMEM((2, blk, f), jnp.float32),
            pltpu.SemaphoreType.DMA((2,)),
        ],
        compiler_params=pltpu.CompilerParams(
            fuse_transposed_lhs_in_matmul=True),
    )(pltpu.with_memory_space_constraint(x, pltpu.HBM), wcol)


def _tc_finish(lin2d, m, s, bias):
    def body(lin_ref, m_ref, s_ref, b_ref, o_ref):
        mm = m_ref[...]
        ss = s_ref[0:mm.shape[0], 0:mm.shape[1]]
        ii = lax.broadcasted_iota(jnp.int32, mm.shape, 0)
        jj = lax.broadcasted_iota(jnp.int32, mm.shape, 1)
        s2 = jnp.sum(jnp.where(jj > ii, mm * ss, 0.0))
        o_ref[...] = jax.nn.sigmoid(lin_ref[...] + (s2 + b_ref[0, 0]))

    return pl.pallas_call(
        body,
        out_shape=jax.ShapeDtypeStruct(lin2d.shape, jnp.float32),
    )(lin2d, m, s, bias)


def kernel(input, fields_dict, W, b, vs):
    fn, f, d = vs.shape
    bsz = input.shape[0]
    vs2 = vs.reshape(fn * f, d)
    fields_pad = jnp.zeros((128,), jnp.int32).at[:f].set(fields_dict)

    m_flat = _sc_pair_dots(vs2, fields_pad)
    m = m_flat[: f * f].reshape(f, f)

    x128 = input @ jnp.eye(f, 128, dtype=jnp.float32)
    w128 = jnp.concatenate(
        [W.reshape(f, 1), jnp.zeros((128 - f, 1), jnp.float32)], axis=0)
    lin2d, s = _tc_stream(x128, w128)
    out2d = _tc_finish(lin2d, m, s, b.reshape(1, 1))
    return out2d.reshape(bsz, 1)
